# baseline (speedup 1.0000x reference)
import jax
import jax.numpy as jnp
from jax import lax
from jax.experimental import pallas as pl
from jax.experimental.pallas import tpu as pltpu
from jax.experimental.pallas import tpu_sc as plsc

_N = 10000
_NPAD = 10240
_E = 320000
_D = 128
_G = 64
_NTILES = 32
_CHUNK = 128
_NCHUNKS = 2560
_CPT = _NCHUNKS // _NTILES
_EPAD = _NCHUNKS * _CHUNK
_RPS = _NPAD // 16
_RPT = _NPAD // _NTILES
_IDXR = 4
_NB = 2

_sc_mesh = plsc.VectorSubcoreMesh(core_axis_name="c", subcore_axis_name="s")


def _scatter_body(h_hbm, sd_hbm, part_hbm, sdidx, rows, acc,
                  isem, gsem, ssem):
    c = lax.axis_index("c")
    s = lax.axis_index("s")
    w = s * 2 + c
    cbase = w * _CPT

    for p in range(_IDXR):
        pltpu.async_copy(sd_hbm.at[cbase + p], sdidx.at[p], isem.at[p])

    zero16 = jnp.zeros((16,), jnp.float32)

    def zrow(i, carry):
        for k in range(8):
            rows[0, i, pl.ds(k * 16, 16)] = zero16
        return carry

    lax.fori_loop(0, _CHUNK, zrow, 0)

    base = s * _RPS
    for off in range(0, _RPS, _CHUNK):
        pltpu.sync_copy(rows.at[0], acc.at[pl.ds(base + off, _CHUNK)])
    plsc.subcore_barrier()

    for p in range(_NB):
        pltpu.make_async_copy(sd_hbm.at[cbase], sdidx.at[p],
                              isem.at[p]).wait()
        pltpu.async_copy(h_hbm.at[sdidx.at[p, 0]], rows.at[p], gsem.at[p])

    def group(g, carry):
        for b in range(_IDXR):
            j = g * _IDXR + b
            rb = b % _NB
            pltpu.make_async_copy(h_hbm.at[sdidx.at[b, 0]], rows.at[rb],
                                  gsem.at[rb]).wait()
            pltpu.async_copy(rows.at[rb], acc.at[sdidx.at[b, 1]],
                             ssem.at[rb], add=True)

            @pl.when(j + _NB < _CPT)
            def _():
                pltpu.make_async_copy(sd_hbm.at[cbase],
                                      sdidx.at[(b + _NB) % _IDXR],
                                      isem.at[(b + _NB) % _IDXR]).wait()
                pltpu.make_async_copy(rows.at[rb], acc.at[sdidx.at[b, 1]],
                                      ssem.at[rb]).wait()
                pltpu.async_copy(h_hbm.at[sdidx.at[(b + _NB) % _IDXR, 0]],
                                 rows.at[rb], gsem.at[rb])

                @pl.when(j + _IDXR < _CPT)
                def _():
                    pltpu.async_copy(sd_hbm.at[cbase + j + _IDXR],
                                     sdidx.at[b], isem.at[b])

        return carry

    lax.fori_loop(0, _CPT // _IDXR, group, 0)
    for b in range(_NB):
        pltpu.make_async_copy(rows.at[b], acc.at[sdidx.at[0, 1]],
                              ssem.at[b]).wait()
    plsc.subcore_barrier()

    for off in range(0, _RPS, _CHUNK):
        pltpu.sync_copy(acc.at[pl.ds(base + off, _CHUNK)],
                        part_hbm.at[c, pl.ds(base + off, _CHUNK)])


_scatter = pl.kernel(
    _scatter_body,
    out_type=jax.ShapeDtypeStruct((2, _NPAD, _D), jnp.float32),
    mesh=_sc_mesh,
    scratch_types=[
        pltpu.VMEM((_IDXR, 2, _CHUNK), jnp.int32),
        pltpu.VMEM((_NB, _CHUNK, _D), jnp.float32),
        pltpu.VMEM_SHARED((_NPAD, _D), jnp.float32),
        pltpu.SemaphoreType.DMA((_IDXR,)),
        pltpu.SemaphoreType.DMA((_NB,)),
        pltpu.SemaphoreType.DMA((_NB,)),
    ],
)


def _pool_body(h_hbm, batch_hbm, pmax_hbm, psum_hbm, pcnt_hbm, hbuf, bbuf,
               tmax, tsum, tcnt):
    c = lax.axis_index("c")
    s = lax.axis_index("s")
    w = s * 2 + c
    base = w * _RPT

    pltpu.sync_copy(h_hbm.at[pl.ds(base, _RPT)], hbuf)
    pltpu.sync_copy(batch_hbm.at[pl.ds(base, _RPT)], bbuf)

    ninf16 = jnp.full((16,), -jnp.inf, jnp.float32)
    zero16 = jnp.zeros((16,), jnp.float32)

    def init_row(g, carry):
        for k in range(8):
            tmax[g, pl.ds(k * 16, 16)] = ninf16
            tsum[g, pl.ds(k * 16, 16)] = zero16
        tcnt[g, pl.ds(0, 16)] = zero16
        return carry

    lax.fori_loop(0, _G, init_row, 0)

    def chunk(j, carry):
        bv = bbuf[pl.ds(j * 16, 16)]
        for l in range(16):
            r = j * 16 + l
            b = bv[l]

            @pl.when(base + r < _N)
            def _():
                for k in range(8):
                    sl = pl.ds(16 * k, 16)
                    hrow = hbuf[r, sl]
                    tmax[b, sl] = jnp.maximum(tmax[b, sl], hrow)
                    tsum[b, sl] = tsum[b, sl] + hrow
                tcnt[b, pl.ds(0, 16)] = tcnt[b, pl.ds(0, 16)] + 1.0

        return carry

    lax.fori_loop(0, _RPT // 16, chunk, 0)

    pltpu.sync_copy(tmax, pmax_hbm.at[w])
    pltpu.sync_copy(tsum, psum_hbm.at[w])
    pltpu.sync_copy(tcnt, pcnt_hbm.at[w])


_pool = pl.kernel(
    _pool_body,
    out_type=(
        jax.ShapeDtypeStruct((_NTILES, _G, _D), jnp.float32),
        jax.ShapeDtypeStruct((_NTILES, _G, _D), jnp.float32),
        jax.ShapeDtypeStruct((_NTILES, _G, 16), jnp.float32),
    ),
    mesh=_sc_mesh,
    scratch_types=[
        pltpu.VMEM((_RPT, _D), jnp.float32),
        pltpu.VMEM((_RPT,), jnp.int32),
        pltpu.VMEM((_G, _D), jnp.float32),
        pltpu.VMEM((_G, _D), jnp.float32),
        pltpu.VMEM((_G, 16), jnp.float32),
    ],
)


def _tc_layer_body(part_ref, hp_ref, wrel_ref, wroot_ref, b_ref, out_ref):
    agg = part_ref[0] + part_ref[1]
    acc = jnp.dot(agg, wrel_ref[...], preferred_element_type=jnp.float32)
    acc = acc + jnp.dot(hp_ref[...], wroot_ref[...],
                        preferred_element_type=jnp.float32)
    out_ref[...] = jnp.maximum(acc + b_ref[...], 0.0)


_tc_layer = pl.pallas_call(
    _tc_layer_body,
    grid=(10,),
    in_specs=[
        pl.BlockSpec((2, 1024, _D), lambda i: (0, i, 0)),
        pl.BlockSpec((1024, _D), lambda i: (i, 0)),
        pl.BlockSpec((_D, _D), lambda i: (0, 0)),
        pl.BlockSpec((_D, _D), lambda i: (0, 0)),
        pl.BlockSpec((1, _D), lambda i: (0, 0)),
    ],
    out_specs=pl.BlockSpec((1024, _D), lambda i: (i, 0)),
    out_shape=jax.ShapeDtypeStruct((_NPAD, _D), jnp.float32),
)


def _final_body(pm1, ps1, pc1, pm2, ps2, pc2, pm3, ps3, pc3,
                w1, d1, w2, d2, w3, d3, logp_ref, enc_ref):
    def pool(pm, ps, pc):
        mx = jnp.max(pm[...], axis=0)
        sm = jnp.sum(ps[...], axis=0)
        cnt = jnp.sum(pc[...], axis=0)[:, 0:1]
        mean = sm / jnp.maximum(cnt, 1.0)
        return jnp.concatenate([mx, mean], axis=1)

    enc = pool(pm1, ps1, pc1) + pool(pm2, ps2, pc2) + pool(pm3, ps3, pc3)
    y = jnp.maximum(
        jnp.dot(enc, w1[...], preferred_element_type=jnp.float32) + d1[...],
        0.0)
    y = jnp.maximum(
        jnp.dot(y, w2[...], preferred_element_type=jnp.float32) + d2[...],
        0.0)
    z = jnp.dot(y, w3[...], preferred_element_type=jnp.float32) + d3[...]
    m = jnp.max(z, axis=1, keepdims=True)
    lse = jnp.log(jnp.sum(jnp.exp(z - m), axis=1, keepdims=True)) + m
    logp_ref[...] = z - lse
    enc_ref[...] = enc


_final = pl.pallas_call(
    _final_body,
    out_shape=(
        jax.ShapeDtypeStruct((_G, 10), jnp.float32),
        jax.ShapeDtypeStruct((_G, 2 * _D), jnp.float32),
    ),
)


def kernel(x, edge_index, batch, Wrel1, Wroot1, b1, Wrel2, Wroot2, b2,
           Wrel3, Wroot3, b3, W1, bl1, W2, bl2, W3, bl3):
    order = jnp.argsort(edge_index[0])
    src = edge_index[0][order]
    dst = edge_index[1][order]
    pad = _EPAD - _E
    pad_dst = _N + (jnp.arange(pad, dtype=jnp.int32) % (_NPAD - _N))
    perm = jnp.arange(_NCHUNKS, dtype=jnp.int32).reshape(_CPT, _NTILES)
    perm = perm.T.reshape(-1)
    srcp = jnp.concatenate([src, jnp.zeros((pad,), jnp.int32)])
    srcp = srcp.reshape(_NCHUNKS, _CHUNK)[perm]
    dstp = jnp.concatenate([dst, pad_dst])
    dstp = dstp.reshape(_NCHUNKS, _CHUNK)[perm]
    sd = jnp.stack([srcp, dstp], axis=1)
    h = jnp.pad(x, ((0, _NPAD - _N), (0, 0)))
    batchp = jnp.pad(batch, (0, _NPAD - _N))

    pools = []
    for Wrel, Wroot, b in ((Wrel1, Wroot1, b1), (Wrel2, Wroot2, b2),
                           (Wrel3, Wroot3, b3)):
        part = _scatter(h, sd)
        h = _tc_layer(part, h, Wrel, Wroot, b.reshape(1, _D))
        pools.append(_pool(h, batchp))

    flat = [t for p in pools for t in p]
    logp, enc = _final(*flat, W1, bl1.reshape(1, -1), W2, bl2.reshape(1, -1),
                       W3, bl3.reshape(1, -1))
    return (logp, jax.lax.stop_gradient(enc))


# speedup vs baseline: 1.5349x; 1.5349x over previous
import jax
import jax.numpy as jnp
from jax import lax
from jax.experimental import pallas as pl
from jax.experimental.pallas import tpu as pltpu
from jax.experimental.pallas import tpu_sc as plsc

_N = 10000
_NPAD = 10240
_E = 320000
_D = 128
_G = 64
_NTILES = 32
_CHUNK = 64
_NCHUNKS = 5120
_CPT = _NCHUNKS // _NTILES
_EPAD = _NCHUNKS * _CHUNK
_RPS = _NPAD // 16
_RPT = _NPAD // _NTILES
_IDXR = 8
_NB = 4

_sc_mesh = plsc.VectorSubcoreMesh(core_axis_name="c", subcore_axis_name="s")


def _scatter_body(h_hbm, sd_hbm, part_hbm, sdidx, rows, acc,
                  isem, gsem, ssem):
    c = lax.axis_index("c")
    s = lax.axis_index("s")
    w = s * 2 + c
    cbase = w * _CPT

    for p in range(_IDXR):
        pltpu.async_copy(sd_hbm.at[cbase + p], sdidx.at[p], isem.at[p])

    zero16 = jnp.zeros((16,), jnp.float32)

    def zrow(i, carry):
        for k in range(8):
            rows[0, i, pl.ds(k * 16, 16)] = zero16
        return carry

    lax.fori_loop(0, _CHUNK, zrow, 0)

    base = s * _RPS
    for off in range(0, _RPS, _CHUNK):
        pltpu.sync_copy(rows.at[0], acc.at[pl.ds(base + off, _CHUNK)])
    plsc.subcore_barrier()

    for p in range(_NB):
        pltpu.make_async_copy(sd_hbm.at[cbase], sdidx.at[p],
                              isem.at[p]).wait()
        pltpu.async_copy(h_hbm.at[sdidx.at[p, 0]], rows.at[p], gsem.at[p])

    def group(g, carry):
        for b in range(_IDXR):
            j = g * _IDXR + b
            rb = b % _NB
            pltpu.make_async_copy(h_hbm.at[sdidx.at[b, 0]], rows.at[rb],
                                  gsem.at[rb]).wait()
            pltpu.async_copy(rows.at[rb], acc.at[sdidx.at[b, 1]],
                             ssem.at[rb], add=True)

            @pl.when(j + _NB < _CPT)
            def _():
                pltpu.make_async_copy(sd_hbm.at[cbase],
                                      sdidx.at[(b + _NB) % _IDXR],
                                      isem.at[(b + _NB) % _IDXR]).wait()
                pltpu.make_async_copy(rows.at[rb], acc.at[sdidx.at[b, 1]],
                                      ssem.at[rb]).wait()
                pltpu.async_copy(h_hbm.at[sdidx.at[(b + _NB) % _IDXR, 0]],
                                 rows.at[rb], gsem.at[rb])

                @pl.when(j + _IDXR < _CPT)
                def _():
                    pltpu.async_copy(sd_hbm.at[cbase + j + _IDXR],
                                     sdidx.at[b], isem.at[b])

        return carry

    lax.fori_loop(0, _CPT // _IDXR, group, 0)
    for b in range(_NB):
        pltpu.make_async_copy(rows.at[b], acc.at[sdidx.at[0, 1]],
                              ssem.at[b]).wait()
    plsc.subcore_barrier()

    for off in range(0, _RPS, _CHUNK):
        pltpu.sync_copy(acc.at[pl.ds(base + off, _CHUNK)],
                        part_hbm.at[c, pl.ds(base + off, _CHUNK)])


_scatter = pl.kernel(
    _scatter_body,
    out_type=jax.ShapeDtypeStruct((2, _NPAD, _D), jnp.float32),
    mesh=_sc_mesh,
    scratch_types=[
        pltpu.VMEM((_IDXR, 2, _CHUNK), jnp.int32),
        pltpu.VMEM((_NB, _CHUNK, _D), jnp.float32),
        pltpu.VMEM_SHARED((_NPAD, _D), jnp.float32),
        pltpu.SemaphoreType.DMA((_IDXR,)),
        pltpu.SemaphoreType.DMA((_NB,)),
        pltpu.SemaphoreType.DMA((_NB,)),
    ],
)


def _pool_body(h_hbm, batch_hbm, pmax_hbm, psum_hbm, pcnt_hbm, hbuf, bbuf,
               tmax, tsum, tcnt):
    c = lax.axis_index("c")
    s = lax.axis_index("s")
    w = s * 2 + c
    base = w * _RPT

    pltpu.sync_copy(h_hbm.at[pl.ds(base, _RPT)], hbuf)
    pltpu.sync_copy(batch_hbm.at[pl.ds(base, _RPT)], bbuf)

    ninf16 = jnp.full((16,), -jnp.inf, jnp.float32)
    zero16 = jnp.zeros((16,), jnp.float32)

    def init_row(g, carry):
        for k in range(8):
            tmax[g, pl.ds(k * 16, 16)] = ninf16
            tsum[g, pl.ds(k * 16, 16)] = zero16
        tcnt[g, pl.ds(0, 16)] = zero16
        return carry

    lax.fori_loop(0, _G, init_row, 0)

    def chunk(j, carry):
        bv = bbuf[pl.ds(j * 16, 16)]
        for l in range(16):
            r = j * 16 + l
            b = bv[l]

            @pl.when(base + r < _N)
            def _():
                for k in range(8):
                    sl = pl.ds(16 * k, 16)
                    hrow = hbuf[r, sl]
                    tmax[b, sl] = jnp.maximum(tmax[b, sl], hrow)
                    tsum[b, sl] = tsum[b, sl] + hrow
                tcnt[b, pl.ds(0, 16)] = tcnt[b, pl.ds(0, 16)] + 1.0

        return carry

    lax.fori_loop(0, _RPT // 16, chunk, 0)

    pltpu.sync_copy(tmax, pmax_hbm.at[w])
    pltpu.sync_copy(tsum, psum_hbm.at[w])
    pltpu.sync_copy(tcnt, pcnt_hbm.at[w])


_pool = pl.kernel(
    _pool_body,
    out_type=(
        jax.ShapeDtypeStruct((_NTILES, _G, _D), jnp.float32),
        jax.ShapeDtypeStruct((_NTILES, _G, _D), jnp.float32),
        jax.ShapeDtypeStruct((_NTILES, _G, 16), jnp.float32),
    ),
    mesh=_sc_mesh,
    scratch_types=[
        pltpu.VMEM((_RPT, _D), jnp.float32),
        pltpu.VMEM((_RPT,), jnp.int32),
        pltpu.VMEM((_G, _D), jnp.float32),
        pltpu.VMEM((_G, _D), jnp.float32),
        pltpu.VMEM((_G, 16), jnp.float32),
    ],
)


def _tc_layer_body(part_ref, hp_ref, wrel_ref, wroot_ref, b_ref, out_ref):
    agg = part_ref[0] + part_ref[1]
    acc = jnp.dot(agg, wrel_ref[...], preferred_element_type=jnp.float32)
    acc = acc + jnp.dot(hp_ref[...], wroot_ref[...],
                        preferred_element_type=jnp.float32)
    out_ref[...] = jnp.maximum(acc + b_ref[...], 0.0)


_tc_layer = pl.pallas_call(
    _tc_layer_body,
    grid=(10,),
    in_specs=[
        pl.BlockSpec((2, 1024, _D), lambda i: (0, i, 0)),
        pl.BlockSpec((1024, _D), lambda i: (i, 0)),
        pl.BlockSpec((_D, _D), lambda i: (0, 0)),
        pl.BlockSpec((_D, _D), lambda i: (0, 0)),
        pl.BlockSpec((1, _D), lambda i: (0, 0)),
    ],
    out_specs=pl.BlockSpec((1024, _D), lambda i: (i, 0)),
    out_shape=jax.ShapeDtypeStruct((_NPAD, _D), jnp.float32),
)


def _final_body(pm1, ps1, pc1, pm2, ps2, pc2, pm3, ps3, pc3,
                w1, d1, w2, d2, w3, d3, logp_ref, enc_ref):
    def pool(pm, ps, pc):
        mx = jnp.max(pm[...], axis=0)
        sm = jnp.sum(ps[...], axis=0)
        cnt = jnp.sum(pc[...], axis=0)[:, 0:1]
        mean = sm / jnp.maximum(cnt, 1.0)
        return jnp.concatenate([mx, mean], axis=1)

    enc = pool(pm1, ps1, pc1) + pool(pm2, ps2, pc2) + pool(pm3, ps3, pc3)
    y = jnp.maximum(
        jnp.dot(enc, w1[...], preferred_element_type=jnp.float32) + d1[...],
        0.0)
    y = jnp.maximum(
        jnp.dot(y, w2[...], preferred_element_type=jnp.float32) + d2[...],
        0.0)
    z = jnp.dot(y, w3[...], preferred_element_type=jnp.float32) + d3[...]
    m = jnp.max(z, axis=1, keepdims=True)
    lse = jnp.log(jnp.sum(jnp.exp(z - m), axis=1, keepdims=True)) + m
    logp_ref[...] = z - lse
    enc_ref[...] = enc


_final = pl.pallas_call(
    _final_body,
    out_shape=(
        jax.ShapeDtypeStruct((_G, 10), jnp.float32),
        jax.ShapeDtypeStruct((_G, 2 * _D), jnp.float32),
    ),
)


def kernel(x, edge_index, batch, Wrel1, Wroot1, b1, Wrel2, Wroot2, b2,
           Wrel3, Wroot3, b3, W1, bl1, W2, bl2, W3, bl3):
    src = edge_index[0]
    dst = edge_index[1]
    pad = _EPAD - _E
    pad_dst = _N + (jnp.arange(pad, dtype=jnp.int32) % (_NPAD - _N))
    perm = jnp.arange(_NCHUNKS, dtype=jnp.int32).reshape(_CPT, _NTILES)
    perm = perm.T.reshape(-1)
    srcp = jnp.concatenate([src, jnp.zeros((pad,), jnp.int32)])
    srcp = srcp.reshape(_NCHUNKS, _CHUNK)[perm]
    dstp = jnp.concatenate([dst, pad_dst])
    dstp = dstp.reshape(_NCHUNKS, _CHUNK)[perm]
    sd = jnp.stack([srcp, dstp], axis=1)
    h = jnp.pad(x, ((0, _NPAD - _N), (0, 0)))
    batchp = jnp.pad(batch, (0, _NPAD - _N))

    pools = []
    for Wrel, Wroot, b in ((Wrel1, Wroot1, b1), (Wrel2, Wroot2, b2),
                           (Wrel3, Wroot3, b3)):
        part = _scatter(h, sd)
        h = _tc_layer(part, h, Wrel, Wroot, b.reshape(1, _D))
        pools.append(_pool(h, batchp))

    flat = [t for p in pools for t in p]
    logp, enc = _final(*flat, W1, bl1.reshape(1, -1), W2, bl2.reshape(1, -1),
                       W3, bl3.reshape(1, -1))
    return (logp, jax.lax.stop_gradient(enc))
